# baseline (speedup 1.0000x reference)
import functools

import jax
import jax.numpy as jnp
from jax import lax
from jax.experimental import pallas as pl
from jax.experimental.pallas import tpu as pltpu
from jax.experimental.pallas import tpu_sc as plsc


def _build(B, L, V, D, NC, NS):
  NW = NC * NS
  BW = B // NW
  SPANS = ((0, 96), (96, L - 96))

  mesh = plsc.VectorSubcoreMesh(core_axis_name="c", subcore_axis_name="s")

  @functools.partial(
      pl.kernel,
      mesh=mesh,
      compiler_params=pltpu.CompilerParams(use_tc_tiling_on_sc=False),
      out_type=jax.ShapeDtypeStruct((B, L, D), jnp.float32),
      scratch_types=[
          pltpu.VMEM((BW, L), jnp.int32),
          pltpu.VMEM((L, D), jnp.float32),
          pltpu.VMEM((L, D), jnp.float32),
          pltpu.VMEM((L, D), jnp.float32),
          pltpu.VMEM((L, D), jnp.float32),
          pltpu.VMEM((L, D), jnp.float32),
          pltpu.SemaphoreType.DMA,
          pltpu.SemaphoreType.DMA,
          pltpu.SemaphoreType.DMA,
          pltpu.SemaphoreType.DMA,
      ],
  )
  def emb(x_hbm, table_hbm, pos_hbm, out_hbm,
          idx_v, pos_v, g0, g1, o0, o1, gs0, gs1, os0, os1):
    bufs, obufs, gsems, osems = [g0, g1], [o0, o1], [gs0, gs1], [os0, os1]
    wid = lax.axis_index("s") * NC + lax.axis_index("c")
    wbase = wid * BW
    pltpu.sync_copy(x_hbm.at[pl.ds(wbase, BW)], idx_v)
    pltpu.sync_copy(pos_hbm, pos_v)

    def gathers(bi, p):
      return [
          pltpu.make_async_copy(
              table_hbm.at[idx_v.at[bi, pl.ds(off, n)]],
              bufs[p].at[pl.ds(off, n)], gsems[p])
          for off, n in SPANS
      ]

    def writeback(bi, p):
      return pltpu.make_async_copy(obufs[p], out_hbm.at[wbase + bi], osems[p])

    for p in range(2):
      for c in gathers(p, p):
        c.start()

    def step(k, carry):
      for p in range(2):
        bi = 2 * k + p
        for c in gathers(bi, p):
          c.wait()
        @pl.when(k > 0)
        def _():
          writeback(bi - 2, p).wait()
        def add_body(r2, c2):
          r = r2 * 2
          for rr in range(2):
            for j in range(D // 16):
              sl = pl.ds(j * 16, 16)
              obufs[p][r + rr, sl] = bufs[p][r + rr, sl] + pos_v[r + rr, sl]
          return c2
        lax.fori_loop(0, L // 2, add_body, 0)
        writeback(bi, p).start()
        @pl.when(k < BW // 2 - 1)
        def _():
          for c in gathers(bi + 2, p):
            c.start()
      return carry

    lax.fori_loop(0, BW // 2, step, 0)
    for p in range(2):
      writeback(BW - 2 + p, p).wait()

  return emb


def kernel(x, token_table, pos_table):
  B, L = x.shape
  V, D = token_table.shape
  info = plsc.get_sparse_core_info()
  NC, NS = info.num_cores, info.num_subcores
  NSPLIT = 4
  BP = B // NSPLIT
  emb = _build(BP, L, V, D, NC, NS)
  xi = x.astype(jnp.int32)
  outs = [emb(xi[i * BP:(i + 1) * BP], token_table, pos_table)
          for i in range(NSPLIT)]
  return jnp.concatenate(outs, axis=0)


# speedup vs baseline: 1.0282x; 1.0282x over previous
import functools

import jax
import jax.numpy as jnp
from jax import lax
from jax.experimental import pallas as pl
from jax.experimental.pallas import tpu as pltpu
from jax.experimental.pallas import tpu_sc as plsc

_NBUF = 4


def _build(B, L, V, D, NC, NS):
  NW = NC * NS
  BW = B // NW
  SPANS = ((0, 96), (96, L - 96))

  mesh = plsc.VectorSubcoreMesh(core_axis_name="c", subcore_axis_name="s")

  @functools.partial(
      pl.kernel,
      mesh=mesh,
      compiler_params=pltpu.CompilerParams(use_tc_tiling_on_sc=False),
      out_type=jax.ShapeDtypeStruct((B, L, D), jnp.float32),
      scratch_types=[
          pltpu.VMEM((BW, L), jnp.int32),
          *[pltpu.VMEM((L, D), jnp.float32) for _ in range(_NBUF)],
          *[pltpu.SemaphoreType.DMA for _ in range(_NBUF)],
      ],
  )
  def emb(x_hbm, table_hbm, out_hbm, idx_v, *bufsems):
    bufs, sems = bufsems[:_NBUF], bufsems[_NBUF:]
    wid = lax.axis_index("s") * NC + lax.axis_index("c")
    wbase = wid * BW
    pltpu.sync_copy(x_hbm.at[pl.ds(wbase, BW)], idx_v)

    def gathers(bi, p):
      return [
          pltpu.make_async_copy(
              table_hbm.at[idx_v.at[bi, pl.ds(off, n)]],
              bufs[p].at[pl.ds(off, n)], sems[p])
          for off, n in SPANS
      ]

    def writeback(bi, p):
      return pltpu.make_async_copy(bufs[p], out_hbm.at[wbase + bi], sems[p])

    for p in range(_NBUF):
      for c in gathers(p, p):
        c.start()

    def step(k, carry):
      for p in range(_NBUF):
        bi = _NBUF * k + p
        for c in gathers(bi, p):
          c.wait()
        writeback(bi, p).start()
        writeback(bi, p).wait()
        @pl.when(k < BW // _NBUF - 1)
        def _():
          for c in gathers(bi + _NBUF, p):
            c.start()
      return carry

    lax.fori_loop(0, BW // _NBUF, step, 0)

  return emb


def kernel(x, token_table, pos_table):
  B, L = x.shape
  V, D = token_table.shape
  info = plsc.get_sparse_core_info()
  NC, NS = info.num_cores, info.num_subcores
  tok = _build(B, L, V, D, NC, NS)(x.astype(jnp.int32), token_table)
  return tok + pos_table[None, :, :]


# speedup vs baseline: 1.2284x; 1.1947x over previous
import functools

import jax
import jax.numpy as jnp
from jax import lax
from jax.experimental import pallas as pl
from jax.experimental.pallas import tpu as pltpu
from jax.experimental.pallas import tpu_sc as plsc


def _build(B, L, V, D, NC, NS):
  NW = NC * NS
  BW = B // NW
  SPANS = ((0, 96), (96, L - 96))

  mesh = plsc.VectorSubcoreMesh(core_axis_name="c", subcore_axis_name="s")

  @functools.partial(
      pl.kernel,
      mesh=mesh,
      compiler_params=pltpu.CompilerParams(use_tc_tiling_on_sc=False),
      out_type=jax.ShapeDtypeStruct((B, L, D), jnp.float32),
      scratch_types=[
          pltpu.VMEM((BW, L), jnp.int32),
          pltpu.VMEM((L, D), jnp.float32),
          pltpu.VMEM((L, D), jnp.float32),
          pltpu.VMEM((L, D), jnp.float32),
          pltpu.VMEM((L, D), jnp.float32),
          pltpu.VMEM((L, D), jnp.float32),
          pltpu.SemaphoreType.DMA,
          pltpu.SemaphoreType.DMA,
          pltpu.SemaphoreType.DMA,
          pltpu.SemaphoreType.DMA,
      ],
  )
  def emb(x_hbm, table_hbm, pos_hbm, out_hbm,
          idx_v, pos_v, g0, g1, o0, o1, gs0, gs1, os0, os1):
    bufs, obufs, gsems, osems = [g0, g1], [o0, o1], [gs0, gs1], [os0, os1]
    wid = lax.axis_index("s") * NC + lax.axis_index("c")
    wbase = wid * BW
    pltpu.sync_copy(x_hbm.at[pl.ds(wbase, BW)], idx_v)
    pltpu.sync_copy(pos_hbm, pos_v)

    def gathers(bi, p):
      return [
          pltpu.make_async_copy(
              table_hbm.at[idx_v.at[bi, pl.ds(off, n)]],
              bufs[p].at[pl.ds(off, n)], gsems[p])
          for off, n in SPANS
      ]

    def writeback(bi, p):
      return pltpu.make_async_copy(obufs[p], out_hbm.at[wbase + bi], osems[p])

    for p in range(2):
      for c in gathers(p, p):
        c.start()

    def step(k, carry):
      for p in range(2):
        bi = 2 * k + p
        for c in gathers(bi, p):
          c.wait()
        @pl.when(k > 0)
        def _():
          writeback(bi - 2, p).wait()
        def add_body(r2, c2):
          r = r2 * 2
          for rr in range(2):
            for j in range(D // 16):
              sl = pl.ds(j * 16, 16)
              obufs[p][r + rr, sl] = bufs[p][r + rr, sl] + pos_v[r + rr, sl]
          return c2
        lax.fori_loop(0, L // 2, add_body, 0)
        writeback(bi, p).start()
        @pl.when(k < BW // 2 - 1)
        def _():
          for c in gathers(bi + 2, p):
            c.start()
      return carry

    lax.fori_loop(0, BW // 2, step, 0)
    for p in range(2):
      writeback(BW - 2 + p, p).wait()

  return emb


def kernel(x, token_table, pos_table):
  B, L = x.shape
  V, D = token_table.shape
  info = plsc.get_sparse_core_info()
  NC, NS = info.num_cores, info.num_subcores
  return _build(B, L, V, D, NC, NS)(
      x.astype(jnp.int32), token_table, pos_table)


# speedup vs baseline: 1.2319x; 1.0028x over previous
import functools

import jax
import jax.numpy as jnp
from jax import lax
from jax.experimental import pallas as pl
from jax.experimental.pallas import tpu as pltpu
from jax.experimental.pallas import tpu_sc as plsc


def _build(B, L, V, D, NC, NS):
  NW = NC * NS
  BW = B // NW
  SPANS = ((0, 96), (96, L - 96))

  mesh = plsc.VectorSubcoreMesh(core_axis_name="c", subcore_axis_name="s")

  @functools.partial(
      pl.kernel,
      mesh=mesh,
      compiler_params=pltpu.CompilerParams(use_tc_tiling_on_sc=False),
      out_type=jax.ShapeDtypeStruct((B, L, D), jnp.float32),
      scratch_types=[
          pltpu.VMEM((BW, L), jnp.int32),
          pltpu.VMEM((L, D), jnp.float32),
          pltpu.VMEM((L, D), jnp.float32),
          pltpu.VMEM((L, D), jnp.float32),
          pltpu.VMEM((L, D), jnp.float32),
          pltpu.VMEM((L, D), jnp.float32),
          pltpu.SemaphoreType.DMA,
          pltpu.SemaphoreType.DMA,
          pltpu.SemaphoreType.DMA,
          pltpu.SemaphoreType.DMA,
      ],
  )
  def emb(x_hbm, table_hbm, pos_hbm, out_hbm,
          idx_v, pos_v, g0, g1, o0, o1, gs0, gs1, os0, os1):
    bufs, obufs, gsems, osems = [g0, g1], [o0, o1], [gs0, gs1], [os0, os1]
    wid = lax.axis_index("s") * NC + lax.axis_index("c")
    wbase = wid * BW
    pltpu.sync_copy(x_hbm.at[pl.ds(wbase, BW)], idx_v)
    pltpu.sync_copy(pos_hbm, pos_v)

    def gathers(bi, p):
      return [
          pltpu.make_async_copy(
              table_hbm.at[idx_v.at[bi, pl.ds(off, n)]],
              bufs[p].at[pl.ds(off, n)], gsems[p])
          for off, n in SPANS
      ]

    def writeback(bi, p):
      return pltpu.make_async_copy(obufs[p], out_hbm.at[wbase + bi], osems[p])

    for p in range(2):
      for c in gathers(p, p):
        c.start()

    def step(k, carry):
      for p in range(2):
        bi = 2 * k + p
        for c in gathers(bi, p):
          c.wait()
        @pl.when(k > 0)
        def _():
          writeback(bi - 2, p).wait()
        def add_body(r4, c2):
          r = r4 * 4
          for rr in range(4):
            for j in range(D // 16):
              sl = pl.ds(j * 16, 16)
              obufs[p][r + rr, sl] = bufs[p][r + rr, sl] + pos_v[r + rr, sl]
          return c2
        lax.fori_loop(0, L // 4, add_body, 0)
        writeback(bi, p).start()
        @pl.when(k < BW // 2 - 1)
        def _():
          for c in gathers(bi + 2, p):
            c.start()
      return carry

    lax.fori_loop(0, BW // 2, step, 0)
    for p in range(2):
      writeback(BW - 2 + p, p).wait()

  return emb


def kernel(x, token_table, pos_table):
  B, L = x.shape
  V, D = token_table.shape
  info = plsc.get_sparse_core_info()
  NC, NS = info.num_cores, info.num_subcores
  return _build(B, L, V, D, NC, NS)(
      x.astype(jnp.int32), token_table, pos_table)
